# Initial kernel scaffold; baseline (speedup 1.0000x reference)
import jax
import jax.numpy as jnp
from jax.experimental import pallas as pl


def kernel(x, wd, wp, gamma, beta):
    raise NotImplementedError("write your pallas kernel here")


# speedup vs baseline: 1.2330x; 1.2330x over previous
import jax
import jax.numpy as jnp
from jax import lax
from jax.experimental import pallas as pl
from jax.experimental.pallas import tpu as pltpu

EPS = 1e-5
THETA = 0.7
KSZ = 3
DIL = 2
PAD = 2
B1 = 8
B2 = 8


def _make_pass1(W, L, B, C):
    taps = []
    for kh in range(KSZ):
        for kw in range(KSZ):
            dh = kh * DIL - PAD
            dw = kw * DIL - PAD
            taps.append((kh * KSZ + kw, dh * W + dw, dh != 0 or dw != 0))

    def body(x_ref, wd_ref, kd_ref, m_ref, cdc_ref, g_ref, v_ref):
        r = jnp.maximum(x_ref[...], 0.0)
        acc = jnp.zeros_like(r)
        for t, off, needs_mask in taps:
            tap = r if off == 0 else pltpu.roll(r, shift=(-off) % L, axis=1)
            if needs_mask:
                tap = tap * m_ref[t:t + 1, :]
            acc = acc + tap * wd_ref[:, t:t + 1]
        cdc = acc - kd_ref[...] * r
        cdc_bf = cdc.astype(jnp.bfloat16)
        cdc_ref[...] = cdc_bf

        g = jnp.zeros((C, C), jnp.float32)
        v = jnp.zeros((C, 1), jnp.float32)
        for b in range(B):
            cb = cdc_bf[b * C:(b + 1) * C, :]
            g = g + lax.dot_general(cb, cb, (((1,), (1,)), ((), ())),
                                    preferred_element_type=jnp.float32)
            v = v + jnp.sum(cdc[b * C:(b + 1) * C, :], axis=1, keepdims=True)
        g_ref[0] = g
        v_ref[0] = v

    return body


def _make_pass2(B, C):
    def body(cdc_ref, wps_ref, sh_ref, o_ref):
        w = wps_ref[...]
        sh = sh_ref[...]
        for b in range(B):
            o_ref[b * C:(b + 1) * C, :] = jnp.dot(
                w, cdc_ref[b * C:(b + 1) * C, :],
                preferred_element_type=jnp.float32) + sh
    return body


def kernel(x, wd, wp, gamma, beta):
    N, C, H, W = x.shape
    Cout = wp.shape[0]
    L = H * W

    x2 = x.reshape(N * C, L)

    wd32 = wd.astype(jnp.float32)
    wd_rows = jnp.tile(wd32.reshape(C, KSZ * KSZ), (B1, 1))
    kd_rows = jnp.tile((THETA * jnp.sum(wd32, axis=(1, 2))).reshape(C, 1),
                       (B1, 1))

    hh = jnp.arange(H).reshape(H, 1)
    ww = jnp.arange(W).reshape(1, W)
    masks = []
    for kh in range(KSZ):
        for kw in range(KSZ):
            dh = kh * DIL - PAD
            dw = kw * DIL - PAD
            m = (hh + dh >= 0) & (hh + dh < H) & (ww + dw >= 0) & (ww + dw < W)
            masks.append(m.reshape(1, L))
    mask_arr = jnp.concatenate(masks, axis=0).astype(jnp.float32)

    n1 = N // B1
    cdc, G, V = pl.pallas_call(
        _make_pass1(W, L, B1, C),
        out_shape=(jax.ShapeDtypeStruct((N * C, L), jnp.bfloat16),
                   jax.ShapeDtypeStruct((n1, C, C), jnp.float32),
                   jax.ShapeDtypeStruct((n1, C, 1), jnp.float32)),
        grid=(n1,),
        in_specs=[pl.BlockSpec((B1 * C, L), lambda i: (i, 0)),
                  pl.BlockSpec((B1 * C, KSZ * KSZ), lambda i: (0, 0)),
                  pl.BlockSpec((B1 * C, 1), lambda i: (0, 0)),
                  pl.BlockSpec((KSZ * KSZ, L), lambda i: (0, 0))],
        out_specs=(pl.BlockSpec((B1 * C, L), lambda i: (i, 0)),
                   pl.BlockSpec((1, C, C), lambda i: (i, 0, 0)),
                   pl.BlockSpec((1, C, 1), lambda i: (i, 0, 0))),
        compiler_params=pltpu.CompilerParams(
            dimension_semantics=("parallel",)),
    )(x2, wd_rows, kd_rows, mask_arr)

    g = jnp.sum(G, axis=0)
    v = jnp.sum(V, axis=0)
    cnt = float(N * L)
    wpf = ((1.0 - THETA) * wp).astype(jnp.float32)
    mean = (wpf @ v) / cnt
    e2 = jnp.sum((wpf @ g) * wpf, axis=1, keepdims=True) / cnt
    var = e2 - mean * mean
    scale = gamma.reshape(Cout, 1).astype(jnp.float32) * lax.rsqrt(var + EPS)
    shift = beta.reshape(Cout, 1).astype(jnp.float32) - mean * scale
    wps = (scale * wpf).astype(jnp.bfloat16)

    n2 = N // B2
    out2 = pl.pallas_call(
        _make_pass2(B2, Cout),
        out_shape=jax.ShapeDtypeStruct((N * Cout, L), jnp.float32),
        grid=(n2,),
        in_specs=[pl.BlockSpec((B2 * C, L), lambda i: (i, 0)),
                  pl.BlockSpec((Cout, C), lambda i: (0, 0)),
                  pl.BlockSpec((Cout, 1), lambda i: (0, 0))],
        out_specs=pl.BlockSpec((B2 * Cout, L), lambda i: (i, 0)),
        compiler_params=pltpu.CompilerParams(
            dimension_semantics=("parallel",)),
    )(cdc, wps, shift)

    return out2.reshape(N, Cout, H, W)


# speedup vs baseline: 1.5820x; 1.2830x over previous
import jax
import jax.numpy as jnp
import numpy as np
from jax import lax
from jax.experimental import pallas as pl
from jax.experimental.pallas import tpu as pltpu

EPS = 1e-5
THETA = 0.7
KSZ = 3
DIL = 2
PAD = 2
CB = 4
NB = 16
NB2 = 16


def _make_pass1(H, W, N, B):
    HW = H * W
    R = B * HW

    def body(x_ref, wd_ref, m_ref, cdc_ref):
        r = jnp.maximum(x_ref[...].reshape(R, N), 0.0)
        t_m = pltpu.roll(r, shift=DIL, axis=0) * m_ref[:, 0:1]
        t_p = pltpu.roll(r, shift=R - DIL, axis=0) * m_ref[:, 1:2]
        groups = []
        for kh in range(KSZ):
            s = (t_m * wd_ref[:, 3 * kh:3 * kh + 1]
                 + r * wd_ref[:, 3 * kh + 1:3 * kh + 2]
                 + t_p * wd_ref[:, 3 * kh + 2:3 * kh + 3])
            groups.append(s)
        cdc = (groups[1]
               + pltpu.roll(groups[0], shift=DIL * W, axis=0) * m_ref[:, 2:3]
               + pltpu.roll(groups[2], shift=R - DIL * W, axis=0) * m_ref[:, 3:4])
        for c in range(B):
            t = jnp.transpose(cdc[c * HW:(c + 1) * HW, :])
            cdc_ref[c] = t.astype(jnp.bfloat16)

    return body


def _make_stats(C, L, B):
    def body(cdc_ref, g_ref, v_ref):
        g = jnp.zeros((C, C), jnp.float32)
        for j in range(B):
            a = cdc_ref[:, j, :].reshape(C, L)
            g = g + lax.dot_general(a, a, (((1,), (1,)), ((), ())),
                                    preferred_element_type=jnp.float32)
        g_ref[0] = g
        v_ref[0] = jnp.sum(cdc_ref[...].astype(jnp.float32),
                           axis=(1, 2)).reshape(C, 1)

    return body


def _make_pass2(C, L, B):
    def body(cdc_ref, wps_ref, sh_ref, o_ref):
        w = wps_ref[...]
        sh = sh_ref[...]
        for j in range(B):
            o_ref[j] = jnp.dot(w, cdc_ref[:, j, :].reshape(C, L),
                               preferred_element_type=jnp.float32) + sh

    return body


def kernel(x, wd, wp, gamma, beta):
    N, C, H, W = x.shape
    Cout = wp.shape[0]
    L = H * W
    HW = L

    xt = x.transpose(1, 2, 3, 0)

    wd32 = wd.astype(jnp.float32)
    wd_flat = wd32.reshape(C, KSZ * KSZ)
    center = (KSZ * KSZ) // 2
    wd_flat = wd_flat.at[:, center].add(-THETA * jnp.sum(wd_flat, axis=1))
    wd_rows = jnp.repeat(wd_flat, HW, axis=0)

    hh = np.arange(H).reshape(H, 1)
    ww = np.arange(W).reshape(1, W)
    mask_np = np.stack([
        np.broadcast_to(ww >= DIL, (H, W)).reshape(HW),
        np.broadcast_to(ww < W - DIL, (H, W)).reshape(HW),
        np.broadcast_to(hh >= DIL, (H, W)).reshape(HW),
        np.broadcast_to(hh < H - DIL, (H, W)).reshape(HW),
    ], axis=1).astype(np.float32)
    mask_arr = jnp.asarray(np.tile(mask_np, (CB, 1)))

    n1 = C // CB
    cdc = pl.pallas_call(
        _make_pass1(H, W, N, CB),
        out_shape=jax.ShapeDtypeStruct((C, N, L), jnp.bfloat16),
        grid=(n1,),
        in_specs=[pl.BlockSpec((CB, H, W, N), lambda i: (i, 0, 0, 0)),
                  pl.BlockSpec((CB * HW, KSZ * KSZ), lambda i: (i, 0)),
                  pl.BlockSpec((CB * HW, 4), lambda i: (0, 0))],
        out_specs=pl.BlockSpec((CB, N, L), lambda i: (i, 0, 0)),
        compiler_params=pltpu.CompilerParams(
            dimension_semantics=("parallel",),
            vmem_limit_bytes=56 * 1024 * 1024),
    )(xt, wd_rows, mask_arr)

    ns = N // NB
    G, V = pl.pallas_call(
        _make_stats(C, L, NB),
        out_shape=(jax.ShapeDtypeStruct((ns, C, C), jnp.float32),
                   jax.ShapeDtypeStruct((ns, C, 1), jnp.float32)),
        grid=(ns,),
        in_specs=[pl.BlockSpec((C, NB, L), lambda i: (0, i, 0))],
        out_specs=(pl.BlockSpec((1, C, C), lambda i: (i, 0, 0)),
                   pl.BlockSpec((1, C, 1), lambda i: (i, 0, 0))),
        compiler_params=pltpu.CompilerParams(
            dimension_semantics=("parallel",)),
    )(cdc)

    g = jnp.sum(G, axis=0)
    v = jnp.sum(V, axis=0)
    cnt = float(N * L)
    wpf = ((1.0 - THETA) * wp).astype(jnp.float32)
    mean = (wpf @ v) / cnt
    e2 = jnp.sum((wpf @ g) * wpf, axis=1, keepdims=True) / cnt
    var = e2 - mean * mean
    scale = gamma.reshape(Cout, 1).astype(jnp.float32) * lax.rsqrt(var + EPS)
    shift = beta.reshape(Cout, 1).astype(jnp.float32) - mean * scale
    wps = (scale * wpf).astype(jnp.bfloat16)

    n2 = N // NB2
    out3 = pl.pallas_call(
        _make_pass2(C, L, NB2),
        out_shape=jax.ShapeDtypeStruct((N, Cout, L), jnp.float32),
        grid=(n2,),
        in_specs=[pl.BlockSpec((C, NB2, L), lambda i: (0, i, 0)),
                  pl.BlockSpec((Cout, C), lambda i: (0, 0)),
                  pl.BlockSpec((Cout, 1), lambda i: (0, 0))],
        out_specs=pl.BlockSpec((NB2, Cout, L), lambda i: (i, 0, 0)),
        compiler_params=pltpu.CompilerParams(
            dimension_semantics=("parallel",)),
    )(cdc, wps, shift)

    return out3.reshape(N, Cout, H, W)


# speedup vs baseline: 2.6236x; 1.6585x over previous
import jax
import jax.numpy as jnp
import numpy as np
from jax import lax
from jax.experimental import pallas as pl
from jax.experimental.pallas import tpu as pltpu

EPS = 1e-5
THETA = 0.7
KSZ = 3
DIL = 2
PAD = 2
B1 = 8
B2 = 8


def _make_pass1(W, L, B, C):
    def body(x_ref, wd_ref, m_ref, cdc_ref, g_ref, v_ref):
        r = jnp.maximum(x_ref[...].reshape(B * C, L), 0.0)
        t_m = pltpu.roll(r, shift=DIL, axis=1) * m_ref[0:1, :]
        t_p = pltpu.roll(r, shift=L - DIL, axis=1) * m_ref[1:2, :]
        groups = []
        for kh in range(KSZ):
            s = (t_m * wd_ref[:, 3 * kh:3 * kh + 1]
                 + r * wd_ref[:, 3 * kh + 1:3 * kh + 2]
                 + t_p * wd_ref[:, 3 * kh + 2:3 * kh + 3])
            groups.append(s)
        cdc = (groups[1]
               + pltpu.roll(groups[0], shift=DIL * W, axis=1) * m_ref[2:3, :]
               + pltpu.roll(groups[2], shift=L - DIL * W, axis=1) * m_ref[3:4, :])
        cdc_bf = cdc.astype(jnp.bfloat16)
        cdc_ref[...] = cdc_bf

        g = jnp.zeros((C, C), jnp.float32)
        for b in range(B):
            cb = cdc_bf[b * C:(b + 1) * C, :]
            g = g + lax.dot_general(cb, cb, (((1,), (1,)), ((), ())),
                                    preferred_element_type=jnp.float32)
        g_ref[0] = g
        v_ref[0] = jnp.sum(cdc.reshape(B, C, L), axis=(0, 2)).reshape(C, 1)

    return body


def _make_pass2(B, C):
    def body(cdc_ref, wps_ref, sh_ref, o_ref):
        w = wps_ref[...]
        sh = sh_ref[...]
        for b in range(B):
            o_ref[b] = jnp.dot(w, cdc_ref[b * C:(b + 1) * C, :],
                               preferred_element_type=jnp.float32) + sh
    return body


def kernel(x, wd, wp, gamma, beta):
    N, C, H, W = x.shape
    Cout = wp.shape[0]
    L = H * W

    wd32 = wd.astype(jnp.float32)
    wd_flat = wd32.reshape(C, KSZ * KSZ)
    onehot = jnp.asarray(
        np.eye(1, KSZ * KSZ, (KSZ * KSZ) // 2, dtype=np.float32))
    wd_flat = wd_flat - (THETA * jnp.sum(wd_flat, axis=1,
                                         keepdims=True)) * onehot
    wd_rows = jnp.tile(wd_flat, (B1, 1))

    hh = np.arange(H).reshape(H, 1)
    ww = np.arange(W).reshape(1, W)
    mask_np = np.stack([
        np.broadcast_to(ww >= DIL, (H, W)).reshape(L),
        np.broadcast_to(ww < W - DIL, (H, W)).reshape(L),
        np.broadcast_to(hh >= DIL, (H, W)).reshape(L),
        np.broadcast_to(hh < H - DIL, (H, W)).reshape(L),
    ]).astype(np.float32)
    mask_arr = jnp.asarray(mask_np)

    n1 = N // B1
    cdc, G, V = pl.pallas_call(
        _make_pass1(W, L, B1, C),
        out_shape=(jax.ShapeDtypeStruct((N * C, L), jnp.bfloat16),
                   jax.ShapeDtypeStruct((n1, C, C), jnp.float32),
                   jax.ShapeDtypeStruct((n1, C, 1), jnp.float32)),
        grid=(n1,),
        in_specs=[pl.BlockSpec((B1, C, L), lambda i: (i, 0, 0)),
                  pl.BlockSpec((B1 * C, KSZ * KSZ), lambda i: (0, 0)),
                  pl.BlockSpec((4, L), lambda i: (0, 0))],
        out_specs=(pl.BlockSpec((B1 * C, L), lambda i: (i, 0)),
                   pl.BlockSpec((1, C, C), lambda i: (i, 0, 0)),
                   pl.BlockSpec((1, C, 1), lambda i: (i, 0, 0))),
        compiler_params=pltpu.CompilerParams(
            dimension_semantics=("parallel",)),
    )(x.reshape(N, C, L), wd_rows, mask_arr)

    g = jnp.sum(G, axis=0)
    v = jnp.sum(V, axis=0)
    cnt = float(N * L)
    wpf = ((1.0 - THETA) * wp).astype(jnp.float32)
    mean = (wpf @ v) / cnt
    e2 = jnp.sum((wpf @ g) * wpf, axis=1, keepdims=True) / cnt
    var = e2 - mean * mean
    scale = gamma.reshape(Cout, 1).astype(jnp.float32) * lax.rsqrt(var + EPS)
    shift = beta.reshape(Cout, 1).astype(jnp.float32) - mean * scale
    wps = (scale * wpf).astype(jnp.bfloat16)

    n2 = N // B2
    out3 = pl.pallas_call(
        _make_pass2(B2, Cout),
        out_shape=jax.ShapeDtypeStruct((N, Cout, L), jnp.float32),
        grid=(n2,),
        in_specs=[pl.BlockSpec((B2 * C, L), lambda i: (i, 0)),
                  pl.BlockSpec((Cout, C), lambda i: (0, 0)),
                  pl.BlockSpec((Cout, 1), lambda i: (0, 0))],
        out_specs=pl.BlockSpec((B2, Cout, L), lambda i: (i, 0, 0)),
        compiler_params=pltpu.CompilerParams(
            dimension_semantics=("parallel",)),
    )(cdc, wps, shift)

    return out3.reshape(N, Cout, H, W)


# speedup vs baseline: 2.6916x; 1.0259x over previous
import jax
import jax.numpy as jnp
import numpy as np
from jax import lax
from jax.experimental import pallas as pl
from jax.experimental.pallas import tpu as pltpu

EPS = 1e-5
THETA = 0.7
KSZ = 3
DIL = 2
PAD = 2
B1 = 16
B2 = 16


def _make_pass1(W, L, B, C):
    def body(x_ref, wd_ref, m_ref, cdc_ref, g_ref, v_ref):
        r = jnp.maximum(x_ref[...].reshape(B * C, L), 0.0)
        t_m = pltpu.roll(r, shift=DIL, axis=1) * m_ref[0:1, :]
        t_p = pltpu.roll(r, shift=L - DIL, axis=1) * m_ref[1:2, :]
        groups = []
        for kh in range(KSZ):
            s = (t_m * wd_ref[:, 3 * kh:3 * kh + 1]
                 + r * wd_ref[:, 3 * kh + 1:3 * kh + 2]
                 + t_p * wd_ref[:, 3 * kh + 2:3 * kh + 3])
            groups.append(s)
        cdc = (groups[1]
               + pltpu.roll(groups[0], shift=DIL * W, axis=1) * m_ref[2:3, :]
               + pltpu.roll(groups[2], shift=L - DIL * W, axis=1) * m_ref[3:4, :])
        cdc_bf = cdc.astype(jnp.bfloat16)
        cdc_ref[...] = cdc_bf

        g = jnp.zeros((C, C), jnp.float32)
        for b in range(B):
            cb = cdc_bf[b * C:(b + 1) * C, :]
            g = g + lax.dot_general(cb, cb, (((1,), (1,)), ((), ())),
                                    preferred_element_type=jnp.float32)
        g_ref[0] = g
        v_ref[0] = jnp.sum(cdc.reshape(B, C, L), axis=(0, 2)).reshape(C, 1)

    return body


def _make_pass2(B, C):
    def body(cdc_ref, wps_ref, sh_ref, o_ref):
        w = wps_ref[...]
        sh = sh_ref[...]
        for b in range(B):
            o_ref[b] = jnp.dot(w, cdc_ref[b * C:(b + 1) * C, :],
                               preferred_element_type=jnp.float32) + sh
    return body


def kernel(x, wd, wp, gamma, beta):
    N, C, H, W = x.shape
    Cout = wp.shape[0]
    L = H * W

    wd32 = wd.astype(jnp.float32)
    wd_flat = wd32.reshape(C, KSZ * KSZ)
    onehot = jnp.asarray(
        np.eye(1, KSZ * KSZ, (KSZ * KSZ) // 2, dtype=np.float32))
    wd_flat = wd_flat - (THETA * jnp.sum(wd_flat, axis=1,
                                         keepdims=True)) * onehot
    wd_rows = jnp.tile(wd_flat, (B1, 1))

    hh = np.arange(H).reshape(H, 1)
    ww = np.arange(W).reshape(1, W)
    mask_np = np.stack([
        np.broadcast_to(ww >= DIL, (H, W)).reshape(L),
        np.broadcast_to(ww < W - DIL, (H, W)).reshape(L),
        np.broadcast_to(hh >= DIL, (H, W)).reshape(L),
        np.broadcast_to(hh < H - DIL, (H, W)).reshape(L),
    ]).astype(np.float32)
    mask_arr = jnp.asarray(mask_np)

    n1 = N // B1
    cdc, G, V = pl.pallas_call(
        _make_pass1(W, L, B1, C),
        out_shape=(jax.ShapeDtypeStruct((N * C, L), jnp.bfloat16),
                   jax.ShapeDtypeStruct((n1, C, C), jnp.float32),
                   jax.ShapeDtypeStruct((n1, C, 1), jnp.float32)),
        grid=(n1,),
        in_specs=[pl.BlockSpec((B1, C, L), lambda i: (i, 0, 0)),
                  pl.BlockSpec((B1 * C, KSZ * KSZ), lambda i: (0, 0)),
                  pl.BlockSpec((4, L), lambda i: (0, 0))],
        out_specs=(pl.BlockSpec((B1 * C, L), lambda i: (i, 0)),
                   pl.BlockSpec((1, C, C), lambda i: (i, 0, 0)),
                   pl.BlockSpec((1, C, 1), lambda i: (i, 0, 0))),
        compiler_params=pltpu.CompilerParams(
            dimension_semantics=("parallel",)),
    )(x.reshape(N, C, L), wd_rows, mask_arr)

    g = jnp.sum(G, axis=0)
    v = jnp.sum(V, axis=0)
    cnt = float(N * L)
    wpf = ((1.0 - THETA) * wp).astype(jnp.float32)
    mean = (wpf @ v) / cnt
    e2 = jnp.sum((wpf @ g) * wpf, axis=1, keepdims=True) / cnt
    var = e2 - mean * mean
    scale = gamma.reshape(Cout, 1).astype(jnp.float32) * lax.rsqrt(var + EPS)
    shift = beta.reshape(Cout, 1).astype(jnp.float32) - mean * scale
    wps = (scale * wpf).astype(jnp.bfloat16)

    n2 = N // B2
    out3 = pl.pallas_call(
        _make_pass2(B2, Cout),
        out_shape=jax.ShapeDtypeStruct((N, Cout, L), jnp.float32),
        grid=(n2,),
        in_specs=[pl.BlockSpec((B2 * C, L), lambda i: (i, 0)),
                  pl.BlockSpec((Cout, C), lambda i: (0, 0)),
                  pl.BlockSpec((Cout, 1), lambda i: (0, 0))],
        out_specs=pl.BlockSpec((B2, Cout, L), lambda i: (i, 0, 0)),
        compiler_params=pltpu.CompilerParams(
            dimension_semantics=("parallel",)),
    )(cdc, wps, shift)

    return out3.reshape(N, Cout, H, W)


# speedup vs baseline: 2.7179x; 1.0098x over previous
import jax
import jax.numpy as jnp
import numpy as np
from jax import lax
from jax.experimental import pallas as pl
from jax.experimental.pallas import tpu as pltpu

EPS = 1e-5
THETA = 0.7
KSZ = 3
DIL = 2
PAD = 2
B1 = 16
B2 = 32


def _make_pass1(W, L, B, C):
    def body(x_ref, wd_ref, m_ref, cdc_ref, g_ref, v_ref):
        r = jnp.maximum(x_ref[...].reshape(B * C, L), 0.0)
        t_m = pltpu.roll(r, shift=DIL, axis=1) * m_ref[0:1, :]
        t_p = pltpu.roll(r, shift=L - DIL, axis=1) * m_ref[1:2, :]
        groups = []
        for kh in range(KSZ):
            s = (t_m * wd_ref[:, 3 * kh:3 * kh + 1]
                 + r * wd_ref[:, 3 * kh + 1:3 * kh + 2]
                 + t_p * wd_ref[:, 3 * kh + 2:3 * kh + 3])
            groups.append(s)
        cdc = (groups[1]
               + pltpu.roll(groups[0], shift=DIL * W, axis=1) * m_ref[2:3, :]
               + pltpu.roll(groups[2], shift=L - DIL * W, axis=1) * m_ref[3:4, :])
        cdc_bf = cdc.astype(jnp.bfloat16)
        cdc_ref[...] = cdc_bf

        g = jnp.zeros((C, C), jnp.float32)
        for b in range(B):
            cb = cdc_bf[b * C:(b + 1) * C, :]
            g = g + lax.dot_general(cb, cb, (((1,), (1,)), ((), ())),
                                    preferred_element_type=jnp.float32)
        g_ref[0] = g
        v_ref[0] = jnp.sum(cdc.reshape(B, C, L), axis=(0, 2)).reshape(C, 1)

    return body


def _make_pass2(B, C):
    def body(cdc_ref, wps_ref, sh_ref, o_ref):
        w = wps_ref[...]
        sh = sh_ref[...]
        for b in range(B):
            o_ref[b] = jnp.dot(w, cdc_ref[b * C:(b + 1) * C, :],
                               preferred_element_type=jnp.float32) + sh
    return body


def kernel(x, wd, wp, gamma, beta):
    N, C, H, W = x.shape
    Cout = wp.shape[0]
    L = H * W

    wd32 = wd.astype(jnp.float32)
    wd_flat = wd32.reshape(C, KSZ * KSZ)
    onehot = jnp.asarray(
        np.eye(1, KSZ * KSZ, (KSZ * KSZ) // 2, dtype=np.float32))
    wd_flat = wd_flat - (THETA * jnp.sum(wd_flat, axis=1,
                                         keepdims=True)) * onehot
    wd_rows = jnp.tile(wd_flat, (B1, 1))

    hh = np.arange(H).reshape(H, 1)
    ww = np.arange(W).reshape(1, W)
    mask_np = np.stack([
        np.broadcast_to(ww >= DIL, (H, W)).reshape(L),
        np.broadcast_to(ww < W - DIL, (H, W)).reshape(L),
        np.broadcast_to(hh >= DIL, (H, W)).reshape(L),
        np.broadcast_to(hh < H - DIL, (H, W)).reshape(L),
    ]).astype(np.float32)
    mask_arr = jnp.asarray(mask_np)

    n1 = N // B1
    cdc, G, V = pl.pallas_call(
        _make_pass1(W, L, B1, C),
        out_shape=(jax.ShapeDtypeStruct((N * C, L), jnp.bfloat16),
                   jax.ShapeDtypeStruct((n1, C, C), jnp.float32),
                   jax.ShapeDtypeStruct((n1, C, 1), jnp.float32)),
        grid=(n1,),
        in_specs=[pl.BlockSpec((B1, C, L), lambda i: (i, 0, 0)),
                  pl.BlockSpec((B1 * C, KSZ * KSZ), lambda i: (0, 0)),
                  pl.BlockSpec((4, L), lambda i: (0, 0))],
        out_specs=(pl.BlockSpec((B1 * C, L), lambda i: (i, 0)),
                   pl.BlockSpec((1, C, C), lambda i: (i, 0, 0)),
                   pl.BlockSpec((1, C, 1), lambda i: (i, 0, 0))),
        compiler_params=pltpu.CompilerParams(
            dimension_semantics=("parallel",)),
    )(x.reshape(N, C, L), wd_rows, mask_arr)

    g = jnp.sum(G, axis=0)
    v = jnp.sum(V, axis=0)
    cnt = float(N * L)
    wpf = ((1.0 - THETA) * wp).astype(jnp.float32)
    mean = (wpf @ v) / cnt
    e2 = jnp.sum((wpf @ g) * wpf, axis=1, keepdims=True) / cnt
    var = e2 - mean * mean
    scale = gamma.reshape(Cout, 1).astype(jnp.float32) * lax.rsqrt(var + EPS)
    shift = beta.reshape(Cout, 1).astype(jnp.float32) - mean * scale
    wps = (scale * wpf).astype(jnp.bfloat16)

    n2 = N // B2
    out3 = pl.pallas_call(
        _make_pass2(B2, Cout),
        out_shape=jax.ShapeDtypeStruct((N, Cout, L), jnp.float32),
        grid=(n2,),
        in_specs=[pl.BlockSpec((B2 * C, L), lambda i: (i, 0)),
                  pl.BlockSpec((Cout, C), lambda i: (0, 0)),
                  pl.BlockSpec((Cout, 1), lambda i: (0, 0))],
        out_specs=pl.BlockSpec((B2, Cout, L), lambda i: (i, 0, 0)),
        compiler_params=pltpu.CompilerParams(
            dimension_semantics=("parallel",)),
    )(cdc, wps, shift)

    return out3.reshape(N, Cout, H, W)
